# baseline (speedup 1.0000x reference)
import functools

import jax
import jax.numpy as jnp
from jax import lax
from jax.experimental import pallas as pl
from jax.experimental.pallas import tpu as pltpu
from jax.experimental.pallas import tpu_sc as plsc

VOCAB = 1000000
EMB = 32
BATCH = 16384

L = 16
NC, NS = 2, 16
NW = NC * NS
BPW = BATCH // NW
CHUNK = 128
NCH = BPW // CHUNK
NGROUPS = BPW // L

_mesh = plsc.VectorSubcoreMesh(core_axis_name="c", subcore_axis_name="s")


@functools.partial(
    pl.kernel,
    out_type=jax.ShapeDtypeStruct((BATCH,), jnp.float32),
    mesh=_mesh,
    compiler_params=pltpu.CompilerParams(needs_layout_passes=False,
                                         use_tc_tiling_on_sc=False),
    scratch_types=[
        pltpu.VMEM((BPW,), jnp.int32),
        pltpu.VMEM((BPW,), jnp.int32),
        pltpu.VMEM((EMB, BPW), jnp.float32),
        pltpu.VMEM((EMB, BPW), jnp.float32),
        pltpu.VMEM((BPW,), jnp.float32),
        pltpu.VMEM((BPW,), jnp.float32),
        pltpu.VMEM((L,), jnp.float32),
        pltpu.VMEM((BPW,), jnp.float32),
        pltpu.SemaphoreType.DMA,
    ],
)
def _mf_kernel(row_id_hbm, col_id_hbm, rembt_hbm, cembt_hbm, rbias_hbm,
               cbias_hbm, gb_hbm, out_hbm,
               ridx_v, cidx_v, rrows_v, crows_v, rb_v, cb_v, gb_v, out_v,
               sem):
    wid = lax.axis_index("s") * NC + lax.axis_index("c")

    pltpu.sync_copy(row_id_hbm.at[wid], ridx_v)
    pltpu.sync_copy(col_id_hbm.at[wid], cidx_v)
    pltpu.sync_copy(gb_hbm, gb_v)

    def gather_g(g, carry):
        sl = pl.ds(g * L, L)
        rid16 = ridx_v[sl]
        cid16 = cidx_v[sl]
        pltpu.async_copy(rbias_hbm.at[rid16], rb_v.at[sl], sem)
        pltpu.async_copy(cbias_hbm.at[cid16], cb_v.at[sl], sem)
        for e in range(EMB):
            pltpu.async_copy(rembt_hbm.at[e].at[rid16], rrows_v.at[e, sl], sem)
            pltpu.async_copy(cembt_hbm.at[e].at[cid16], crows_v.at[e, sl], sem)
        return carry

    lax.fori_loop(0, NGROUPS, gather_g, 0)

    pltpu.make_async_copy(rbias_hbm, rb_v, sem).wait()
    pltpu.make_async_copy(cbias_hbm, cb_v, sem).wait()
    pltpu.make_async_copy(rembt_hbm, rrows_v, sem).wait()
    pltpu.make_async_copy(cembt_hbm, crows_v, sem).wait()

    gbv = gb_v[...]

    def group(g, carry):
        sl = pl.ds(g * L, L)
        acc = rb_v[sl] + cb_v[sl] + gbv
        for e in range(EMB):
            acc = acc + rrows_v[e, sl] * crows_v[e, sl]
        out_v[sl] = acc
        return carry

    lax.fori_loop(0, NGROUPS, group, 0)
    pltpu.sync_copy(out_v, out_hbm.at[pl.ds(wid * BPW, BPW)])


def kernel(row_id, col_id, row_emb_table, col_emb_table, row_bias_table,
           col_bias_table, global_bias):
    rid = row_id.astype(jnp.int32).reshape(NW, BPW)
    cid = col_id.astype(jnp.int32).reshape(NW, BPW)
    rembt = row_emb_table.T
    cembt = col_emb_table.T
    rb = row_bias_table.reshape(VOCAB)
    cb = col_bias_table.reshape(VOCAB)
    gb = jnp.broadcast_to(global_bias.astype(jnp.float32), (L,))
    out = _mf_kernel(rid, cid, rembt, cembt, rb, cb, gb)
    return out.reshape(BATCH, 1)
